# baseline (speedup 1.0000x reference)
import functools

import jax
import jax.numpy as jnp
from jax import lax
from jax.experimental import pallas as pl
from jax.experimental.pallas import tpu as pltpu
from jax.experimental.pallas import tpu_sc as plsc

NC = 2
NS = 16
C = 64
K = 3
ZR = 200


def _sc_fused_step(table, src, dst, h, qzero):
    e, d = h.shape
    n = table.shape[0]
    eh = e // 2
    n_chunks = eh // C
    nloop = -(-n_chunks // NS)
    rounds = -(-nloop // K)
    z_chunks = n // ZR
    z_loop = -(-z_chunks // NS)

    scratch = ([pltpu.VMEM((C,), jnp.int32) for _ in range(2 * K)]
               + [pltpu.VMEM((C, d), jnp.float32) for _ in range(2 * K)]
               + [pltpu.VMEM_SHARED((n, d), jnp.float32)]
               + [pltpu.SemaphoreType.DMA for _ in range(5 * K)])

    @functools.partial(
        pl.kernel,
        out_type=[jax.ShapeDtypeStruct((e, d), jnp.float32),
                  jax.ShapeDtypeStruct((NC, n, d), jnp.float32)],
        mesh=plsc.VectorSubcoreMesh(core_axis_name="c", subcore_axis_name="s"),
        scratch_types=scratch,
    )
    def k(table_hbm, src_hbm, dst_hbm, h_hbm, qz_hbm, out_hbm, a_hbm, *sc):
        isrc = sc[0:K]
        idst = sc[K:2 * K]
        gbuf = sc[2 * K:3 * K]
        hbuf = sc[3 * K:4 * K]
        a_sh = sc[4 * K]
        sem_s = sc[4 * K + 1:5 * K + 1]
        sem_d = sc[5 * K + 1:6 * K + 1]
        sem_g = sc[6 * K + 1:7 * K + 1]
        sem_h = sc[7 * K + 1:8 * K + 1]
        sem_o = sc[8 * K + 1:9 * K + 1]
        cid = lax.axis_index("c")
        sid = lax.axis_index("s")
        ebase = cid * eh

        def zero(i, carry):
            chunk = sid + i * NS

            @pl.when(chunk < z_chunks)
            def _():
                pltpu.sync_copy(qz_hbm.at[pl.ds(chunk * ZR, ZR)],
                                a_sh.at[pl.ds(chunk * ZR, ZR)])

            return carry

        lax.fori_loop(0, z_loop, zero, 0)
        plsc.subcore_barrier()

        def start_in(b, j):
            @pl.when(sid + j * NS < n_chunks)
            def _():
                base = ebase + (sid + j * NS) * C
                pltpu.async_copy(src_hbm.at[pl.ds(base, C)], isrc[b], sem_s[b])
                pltpu.async_copy(dst_hbm.at[pl.ds(base, C)], idst[b], sem_d[b])
                pltpu.async_copy(h_hbm.at[pl.ds(base, C)], hbuf[b], sem_h[b])

        for b in range(K):
            start_in(b, b)

        def rnd(r, carry):
            for b in range(K):
                j = r * K + b

                @pl.when(sid + j * NS < n_chunks)
                def _():
                    base = ebase + (sid + j * NS) * C
                    pltpu.make_async_copy(src_hbm.at[pl.ds(base, C)],
                                          isrc[b], sem_s[b]).wait()
                    pltpu.async_copy(table_hbm.at[isrc[b]], gbuf[b], sem_g[b])
                    pltpu.make_async_copy(h_hbm.at[pl.ds(base, C)],
                                          hbuf[b], sem_h[b]).wait()
                    pltpu.make_async_copy(table_hbm.at[isrc[b]],
                                          gbuf[b], sem_g[b]).wait()

                    def upd(rr, carry2):
                        for c8 in range(d // 16):
                            sl = pl.ds(c8 * 16, 16)
                            gbuf[b][rr, sl] = jnp.maximum(
                                gbuf[b][rr, sl] + hbuf[b][rr, sl], 0.0)
                        return carry2

                    lax.fori_loop(0, C, upd, 0)
                    pltpu.async_copy(gbuf[b], out_hbm.at[pl.ds(base, C)],
                                     sem_o[b])
                    pltpu.make_async_copy(dst_hbm.at[pl.ds(base, C)],
                                          idst[b], sem_d[b]).wait()
                    pltpu.sync_copy(gbuf[b], a_sh.at[idst[b]], add=True)
                    pltpu.make_async_copy(gbuf[b], out_hbm.at[pl.ds(base, C)],
                                          sem_o[b]).wait()
                    start_in(b, j + K)

            return carry

        lax.fori_loop(0, rounds, rnd, 0)
        plsc.subcore_barrier()

        def wb(i, carry):
            chunk = sid + i * NS

            @pl.when(chunk < z_chunks)
            def _():
                pltpu.sync_copy(a_sh.at[pl.ds(chunk * ZR, ZR)],
                                a_hbm.at[cid, pl.ds(chunk * ZR, ZR)])

            return carry

        lax.fori_loop(0, z_loop, wb, 0)

    return k(table, src, dst, h, qzero)


def _tc_matmul(x, w, block_rows):
    m, kdim = x.shape
    _, nout = w.shape

    def body(x_ref, w_ref, o_ref):
        o_ref[...] = jnp.dot(x_ref[...], w_ref[...],
                             preferred_element_type=jnp.float32)

    return pl.pallas_call(
        body,
        grid=(m // block_rows,),
        in_specs=[pl.BlockSpec((block_rows, kdim), lambda i: (i, 0)),
                  pl.BlockSpec((kdim, nout), lambda i: (0, 0))],
        out_specs=pl.BlockSpec((block_rows, nout), lambda i: (i, 0)),
        out_shape=jax.ShapeDtypeStruct((m, nout), jnp.float32),
    )(x, w)


def _tc_h0(efeat, w1b, b_init, block_rows):
    e, de = efeat.shape
    d = w1b.shape[1]

    def body(ef_ref, w_ref, b_ref, o_ref):
        o_ref[...] = jnp.dot(ef_ref[...], w_ref[...],
                             preferred_element_type=jnp.float32) + b_ref[...]

    return pl.pallas_call(
        body,
        grid=(e // block_rows,),
        in_specs=[pl.BlockSpec((block_rows, de), lambda i: (i, 0)),
                  pl.BlockSpec((de, d), lambda i: (0, 0)),
                  pl.BlockSpec((1, d), lambda i: (0, 0))],
        out_specs=pl.BlockSpec((block_rows, d), lambda i: (i, 0)),
        out_shape=jax.ShapeDtypeStruct((e, d), jnp.float32),
    )(efeat, w1b, b_init)


def _tc_h(ef, w_upd, b_upd, block_rows):
    e, d = ef.shape
    grid = e // block_rows
    hb = grid // 2

    def body(efa_ref, efb_ref, b_ref, w_ref, o_ref):
        efa = efa_ref[...]
        efb = efb_ref[...]
        o_ref[0, ...] = efa + b_ref[...] - jnp.dot(
            efb, w_ref[...], preferred_element_type=jnp.float32)
        o_ref[1, ...] = efb + b_ref[...] - jnp.dot(
            efa, w_ref[...], preferred_element_type=jnp.float32)

    out = pl.pallas_call(
        body,
        grid=(hb,),
        in_specs=[pl.BlockSpec((block_rows, d), lambda i: (i, 0)),
                  pl.BlockSpec((block_rows, d), lambda i: (i + hb, 0)),
                  pl.BlockSpec((1, d), lambda i: (0, 0)),
                  pl.BlockSpec((d, d), lambda i: (0, 0))],
        out_specs=pl.BlockSpec((2, block_rows, d), lambda i: (0, i, 0)),
        out_shape=jax.ShapeDtypeStruct((2, e // 2, d), jnp.float32),
    )(ef, ef, b_upd, w_upd)
    return out.reshape(e, d)


def _tc_qcomb(a0, a1, w_upd, block_rows):
    n, d = a0.shape

    def body(a0_ref, a1_ref, w_ref, o_ref):
        o_ref[...] = jnp.dot(a0_ref[...] + a1_ref[...], w_ref[...],
                             preferred_element_type=jnp.float32)

    return pl.pallas_call(
        body,
        grid=(n // block_rows,),
        in_specs=[pl.BlockSpec((block_rows, d), lambda i: (i, 0)),
                  pl.BlockSpec((block_rows, d), lambda i: (i, 0)),
                  pl.BlockSpec((d, d), lambda i: (0, 0))],
        out_specs=pl.BlockSpec((block_rows, d), lambda i: (i, 0)),
        out_shape=jax.ShapeDtypeStruct((n, d), jnp.float32),
    )(a0, a1, w_upd)


def _tc_final(nf, m0, m1, wfa, wfb, b_fin, block_rows):
    n, d = nf.shape

    def body(nf_ref, m0_ref, m1_ref, wa_ref, wb_ref, b_ref, o_ref):
        acc = jnp.dot(nf_ref[...], wa_ref[...], preferred_element_type=jnp.float32)
        acc += jnp.dot(m0_ref[...] + m1_ref[...], wb_ref[...],
                       preferred_element_type=jnp.float32)
        o_ref[...] = jnp.maximum(acc + b_ref[...], 0.0)

    return pl.pallas_call(
        body,
        grid=(n // block_rows,),
        in_specs=[pl.BlockSpec((block_rows, d), lambda i: (i, 0)),
                  pl.BlockSpec((block_rows, d), lambda i: (i, 0)),
                  pl.BlockSpec((block_rows, d), lambda i: (i, 0)),
                  pl.BlockSpec((d, d), lambda i: (0, 0)),
                  pl.BlockSpec((d, d), lambda i: (0, 0)),
                  pl.BlockSpec((1, d), lambda i: (0, 0))],
        out_specs=pl.BlockSpec((block_rows, d), lambda i: (i, 0)),
        out_shape=jax.ShapeDtypeStruct((n, d), jnp.float32),
    )(nf, m0, m1, wfa, wfb, b_fin)


STEPS = 4
BLOCK_E = 640
BLOCK_N = 2000


def kernel(node_feature, edge_feature, W_init, b_init, W_upd, b_upd,
           W_fin, b_fin, edge_src, edge_dst):
    n, d = node_feature.shape

    w1a, w1b = W_init[:d], W_init[d:]
    wfa, wfb = W_fin[:d], W_fin[d:]
    b_init2 = b_init.reshape(1, -1)
    b_upd2 = b_upd.reshape(1, -1)
    b_fin2 = b_fin.reshape(1, -1)
    qzero = jnp.zeros((n, d), jnp.float32)

    p = _tc_matmul(node_feature, w1a, BLOCK_N)
    h0 = _tc_h0(edge_feature, w1b, b_init2, BLOCK_E)
    ef, parts = _sc_fused_step(p, edge_src, edge_dst, h0, qzero)

    for _ in range(STEPS):
        q = _tc_qcomb(parts[0], parts[1], W_upd, BLOCK_N)
        hh = _tc_h(ef, W_upd, b_upd2, BLOCK_E)
        ef, parts = _sc_fused_step(q, edge_src, edge_dst, hh, qzero)

    return _tc_final(node_feature, parts[0], parts[1], wfa, wfb, b_fin2, BLOCK_N)


# speedup vs baseline: 1.0110x; 1.0110x over previous
import functools

import jax
import jax.numpy as jnp
from jax import lax
from jax.experimental import pallas as pl
from jax.experimental.pallas import tpu as pltpu
from jax.experimental.pallas import tpu_sc as plsc

NC = 2
NS = 16
C = 64
K = 3
ZR = 200


def _sc_fused_step(table, src, dst, h, qzero):
    e, d = h.shape
    n = table.shape[0]
    eh = e // 2
    n_chunks = eh // C
    nloop = -(-n_chunks // NS)
    rounds = -(-nloop // K)
    z_chunks = n // ZR
    z_loop = -(-z_chunks // NS)

    scratch = ([pltpu.VMEM((C,), jnp.int32) for _ in range(2 * K)]
               + [pltpu.VMEM((C, d), jnp.float32) for _ in range(2 * K)]
               + [pltpu.VMEM_SHARED((n, d), jnp.float32)]
               + [pltpu.SemaphoreType.DMA for _ in range(6 * K)])

    @functools.partial(
        pl.kernel,
        out_type=[jax.ShapeDtypeStruct((e, d), jnp.float32),
                  jax.ShapeDtypeStruct((NC, n, d), jnp.float32)],
        mesh=plsc.VectorSubcoreMesh(core_axis_name="c", subcore_axis_name="s"),
        scratch_types=scratch,
    )
    def k(table_hbm, src_hbm, dst_hbm, h_hbm, qz_hbm, out_hbm, a_hbm, *sc):
        isrc = sc[0:K]
        idst = sc[K:2 * K]
        gbuf = sc[2 * K:3 * K]
        hbuf = sc[3 * K:4 * K]
        a_sh = sc[4 * K]
        sem_s = sc[4 * K + 1:5 * K + 1]
        sem_d = sc[5 * K + 1:6 * K + 1]
        sem_g = sc[6 * K + 1:7 * K + 1]
        sem_h = sc[7 * K + 1:8 * K + 1]
        sem_o = sc[8 * K + 1:9 * K + 1]
        sem_a = sc[9 * K + 1:10 * K + 1]
        cid = lax.axis_index("c")
        sid = lax.axis_index("s")
        ebase = cid * eh

        def zero(i, carry):
            chunk = sid + i * NS

            @pl.when(chunk < z_chunks)
            def _():
                pltpu.sync_copy(qz_hbm.at[pl.ds(chunk * ZR, ZR)],
                                a_sh.at[pl.ds(chunk * ZR, ZR)])

            return carry

        lax.fori_loop(0, z_loop, zero, 0)
        plsc.subcore_barrier()

        def start_in(b, j):
            @pl.when(sid + j * NS < n_chunks)
            def _():
                base = ebase + (sid + j * NS) * C
                pltpu.async_copy(src_hbm.at[pl.ds(base, C)], isrc[b], sem_s[b])
                pltpu.async_copy(dst_hbm.at[pl.ds(base, C)], idst[b], sem_d[b])
                pltpu.async_copy(h_hbm.at[pl.ds(base, C)], hbuf[b], sem_h[b])

        for b in range(K):
            start_in(b, b)

        def rnd(r, carry):
            for b in range(K):
                j = r * K + b

                @pl.when((j >= K) & (sid + (j - K) * NS < n_chunks))
                def _():
                    pltpu.make_async_copy(gbuf[b], a_sh.at[idst[b]],
                                          sem_a[b]).wait()

                @pl.when(sid + j * NS < n_chunks)
                def _():
                    base = ebase + (sid + j * NS) * C
                    pltpu.make_async_copy(src_hbm.at[pl.ds(base, C)],
                                          isrc[b], sem_s[b]).wait()
                    pltpu.async_copy(table_hbm.at[isrc[b]], gbuf[b], sem_g[b])
                    pltpu.make_async_copy(h_hbm.at[pl.ds(base, C)],
                                          hbuf[b], sem_h[b]).wait()
                    pltpu.make_async_copy(table_hbm.at[isrc[b]],
                                          gbuf[b], sem_g[b]).wait()

                    def upd(rr, carry2):
                        for c8 in range(d // 16):
                            sl = pl.ds(c8 * 16, 16)
                            gbuf[b][rr, sl] = jnp.maximum(
                                gbuf[b][rr, sl] + hbuf[b][rr, sl], 0.0)
                        return carry2

                    lax.fori_loop(0, C, upd, 0)
                    pltpu.async_copy(gbuf[b], out_hbm.at[pl.ds(base, C)],
                                     sem_o[b])
                    pltpu.make_async_copy(dst_hbm.at[pl.ds(base, C)],
                                          idst[b], sem_d[b]).wait()
                    pltpu.async_copy(gbuf[b], a_sh.at[idst[b]], sem_a[b],
                                     add=True)
                    pltpu.make_async_copy(gbuf[b], out_hbm.at[pl.ds(base, C)],
                                          sem_o[b]).wait()
                    start_in(b, j + K)

            return carry

        lax.fori_loop(0, rounds, rnd, 0)

        for b in range(K):
            jl = (rounds - 1) * K + b

            @pl.when(sid + jl * NS < n_chunks)
            def _():
                pltpu.make_async_copy(gbuf[b], a_sh.at[idst[b]],
                                      sem_a[b]).wait()

        plsc.subcore_barrier()

        def wb(i, carry):
            chunk = sid + i * NS

            @pl.when(chunk < z_chunks)
            def _():
                pltpu.sync_copy(a_sh.at[pl.ds(chunk * ZR, ZR)],
                                a_hbm.at[cid, pl.ds(chunk * ZR, ZR)])

            return carry

        lax.fori_loop(0, z_loop, wb, 0)

    return k(table, src, dst, h, qzero)


def _tc_matmul(x, w, block_rows):
    m, kdim = x.shape
    _, nout = w.shape

    def body(x_ref, w_ref, o_ref):
        o_ref[...] = jnp.dot(x_ref[...], w_ref[...],
                             preferred_element_type=jnp.float32)

    return pl.pallas_call(
        body,
        grid=(m // block_rows,),
        in_specs=[pl.BlockSpec((block_rows, kdim), lambda i: (i, 0)),
                  pl.BlockSpec((kdim, nout), lambda i: (0, 0))],
        out_specs=pl.BlockSpec((block_rows, nout), lambda i: (i, 0)),
        out_shape=jax.ShapeDtypeStruct((m, nout), jnp.float32),
    )(x, w)


def _tc_h0(efeat, w1b, b_init, block_rows):
    e, de = efeat.shape
    d = w1b.shape[1]

    def body(ef_ref, w_ref, b_ref, o_ref):
        o_ref[...] = jnp.dot(ef_ref[...], w_ref[...],
                             preferred_element_type=jnp.float32) + b_ref[...]

    return pl.pallas_call(
        body,
        grid=(e // block_rows,),
        in_specs=[pl.BlockSpec((block_rows, de), lambda i: (i, 0)),
                  pl.BlockSpec((de, d), lambda i: (0, 0)),
                  pl.BlockSpec((1, d), lambda i: (0, 0))],
        out_specs=pl.BlockSpec((block_rows, d), lambda i: (i, 0)),
        out_shape=jax.ShapeDtypeStruct((e, d), jnp.float32),
    )(efeat, w1b, b_init)


def _tc_h(ef, w_upd, b_upd, block_rows):
    e, d = ef.shape
    grid = e // block_rows
    hb = grid // 2

    def body(efa_ref, efb_ref, b_ref, w_ref, o_ref):
        efa = efa_ref[...]
        efb = efb_ref[...]
        o_ref[0, ...] = efa + b_ref[...] - jnp.dot(
            efb, w_ref[...], preferred_element_type=jnp.float32)
        o_ref[1, ...] = efb + b_ref[...] - jnp.dot(
            efa, w_ref[...], preferred_element_type=jnp.float32)

    out = pl.pallas_call(
        body,
        grid=(hb,),
        in_specs=[pl.BlockSpec((block_rows, d), lambda i: (i, 0)),
                  pl.BlockSpec((block_rows, d), lambda i: (i + hb, 0)),
                  pl.BlockSpec((1, d), lambda i: (0, 0)),
                  pl.BlockSpec((d, d), lambda i: (0, 0))],
        out_specs=pl.BlockSpec((2, block_rows, d), lambda i: (0, i, 0)),
        out_shape=jax.ShapeDtypeStruct((2, e // 2, d), jnp.float32),
    )(ef, ef, b_upd, w_upd)
    return out.reshape(e, d)


def _tc_qcomb(a0, a1, w_upd, block_rows):
    n, d = a0.shape

    def body(a0_ref, a1_ref, w_ref, o_ref):
        o_ref[...] = jnp.dot(a0_ref[...] + a1_ref[...], w_ref[...],
                             preferred_element_type=jnp.float32)

    return pl.pallas_call(
        body,
        grid=(n // block_rows,),
        in_specs=[pl.BlockSpec((block_rows, d), lambda i: (i, 0)),
                  pl.BlockSpec((block_rows, d), lambda i: (i, 0)),
                  pl.BlockSpec((d, d), lambda i: (0, 0))],
        out_specs=pl.BlockSpec((block_rows, d), lambda i: (i, 0)),
        out_shape=jax.ShapeDtypeStruct((n, d), jnp.float32),
    )(a0, a1, w_upd)


def _tc_final(nf, m0, m1, wfa, wfb, b_fin, block_rows):
    n, d = nf.shape

    def body(nf_ref, m0_ref, m1_ref, wa_ref, wb_ref, b_ref, o_ref):
        acc = jnp.dot(nf_ref[...], wa_ref[...], preferred_element_type=jnp.float32)
        acc += jnp.dot(m0_ref[...] + m1_ref[...], wb_ref[...],
                       preferred_element_type=jnp.float32)
        o_ref[...] = jnp.maximum(acc + b_ref[...], 0.0)

    return pl.pallas_call(
        body,
        grid=(n // block_rows,),
        in_specs=[pl.BlockSpec((block_rows, d), lambda i: (i, 0)),
                  pl.BlockSpec((block_rows, d), lambda i: (i, 0)),
                  pl.BlockSpec((block_rows, d), lambda i: (i, 0)),
                  pl.BlockSpec((d, d), lambda i: (0, 0)),
                  pl.BlockSpec((d, d), lambda i: (0, 0)),
                  pl.BlockSpec((1, d), lambda i: (0, 0))],
        out_specs=pl.BlockSpec((block_rows, d), lambda i: (i, 0)),
        out_shape=jax.ShapeDtypeStruct((n, d), jnp.float32),
    )(nf, m0, m1, wfa, wfb, b_fin)


STEPS = 4
BLOCK_E = 640
BLOCK_N = 2000


def kernel(node_feature, edge_feature, W_init, b_init, W_upd, b_upd,
           W_fin, b_fin, edge_src, edge_dst):
    n, d = node_feature.shape

    w1a, w1b = W_init[:d], W_init[d:]
    wfa, wfb = W_fin[:d], W_fin[d:]
    b_init2 = b_init.reshape(1, -1)
    b_upd2 = b_upd.reshape(1, -1)
    b_fin2 = b_fin.reshape(1, -1)
    qzero = jnp.zeros((n, d), jnp.float32)

    p = _tc_matmul(node_feature, w1a, BLOCK_N)
    h0 = _tc_h0(edge_feature, w1b, b_init2, BLOCK_E)
    ef, parts = _sc_fused_step(p, edge_src, edge_dst, h0, qzero)

    for _ in range(STEPS):
        q = _tc_qcomb(parts[0], parts[1], W_upd, BLOCK_N)
        hh = _tc_h(ef, W_upd, b_upd2, BLOCK_E)
        ef, parts = _sc_fused_step(q, edge_src, edge_dst, hh, qzero)

    return _tc_final(node_feature, parts[0], parts[1], wfa, wfb, b_fin2, BLOCK_N)


# speedup vs baseline: 1.0249x; 1.0138x over previous
import functools

import jax
import jax.numpy as jnp
from jax import lax
from jax.experimental import pallas as pl
from jax.experimental.pallas import tpu as pltpu
from jax.experimental.pallas import tpu_sc as plsc

NC = 2
NS = 16
C = 64
K = 3
ZR = 200


def _sc_fused_step(table, src, dst, h, qzero):
    e, d = h.shape
    n = table.shape[0]
    eh = e // 2
    n_chunks = eh // C
    nloop = -(-n_chunks // NS)
    rounds = -(-nloop // K)
    z_chunks = n // ZR
    z_loop = -(-z_chunks // NS)

    scratch = ([pltpu.VMEM((C,), jnp.int32) for _ in range(2 * K)]
               + [pltpu.VMEM((C, d), jnp.float32) for _ in range(2 * K)]
               + [pltpu.VMEM_SHARED((n, d), jnp.float32)]
               + [pltpu.SemaphoreType.DMA for _ in range(6 * K)])

    @functools.partial(
        pl.kernel,
        out_type=[jax.ShapeDtypeStruct((e, d), jnp.float32),
                  jax.ShapeDtypeStruct((NC, n, d), jnp.float32)],
        mesh=plsc.VectorSubcoreMesh(core_axis_name="c", subcore_axis_name="s"),
        scratch_types=scratch,
    )
    def k(table_hbm, src_hbm, dst_hbm, h_hbm, qz_hbm, out_hbm, a_hbm, *sc):
        isrc = sc[0:K]
        idst = sc[K:2 * K]
        gbuf = sc[2 * K:3 * K]
        hbuf = sc[3 * K:4 * K]
        a_sh = sc[4 * K]
        sem_s = sc[4 * K + 1:5 * K + 1]
        sem_d = sc[5 * K + 1:6 * K + 1]
        sem_g = sc[6 * K + 1:7 * K + 1]
        sem_h = sc[7 * K + 1:8 * K + 1]
        sem_o = sc[8 * K + 1:9 * K + 1]
        sem_a = sc[9 * K + 1:10 * K + 1]
        cid = lax.axis_index("c")
        sid = lax.axis_index("s")
        ebase = cid * eh

        def zero(i, carry):
            chunk = sid + i * NS

            @pl.when(chunk < z_chunks)
            def _():
                pltpu.sync_copy(qz_hbm.at[pl.ds(chunk * ZR, ZR)],
                                a_sh.at[pl.ds(chunk * ZR, ZR)])

            return carry

        lax.fori_loop(0, z_loop, zero, 0)
        plsc.subcore_barrier()

        def start_in(b, j):
            @pl.when(sid + j * NS < n_chunks)
            def _():
                base = ebase + (sid + j * NS) * C
                pltpu.async_copy(src_hbm.at[pl.ds(base, C)], isrc[b], sem_s[b])
                pltpu.async_copy(dst_hbm.at[pl.ds(base, C)], idst[b], sem_d[b])
                pltpu.async_copy(h_hbm.at[pl.ds(base, C)], hbuf[b], sem_h[b])

        for b in range(K):
            start_in(b, b)

        def rnd(r, carry):
            for b in range(K):
                j = r * K + b

                @pl.when((j >= K) & (sid + (j - K) * NS < n_chunks))
                def _():
                    pltpu.make_async_copy(gbuf[b], a_sh.at[idst[b]],
                                          sem_a[b]).wait()

                @pl.when(sid + j * NS < n_chunks)
                def _():
                    base = ebase + (sid + j * NS) * C
                    pltpu.make_async_copy(src_hbm.at[pl.ds(base, C)],
                                          isrc[b], sem_s[b]).wait()
                    pltpu.async_copy(table_hbm.at[isrc[b]], gbuf[b], sem_g[b])
                    pltpu.make_async_copy(h_hbm.at[pl.ds(base, C)],
                                          hbuf[b], sem_h[b]).wait()
                    pltpu.make_async_copy(table_hbm.at[isrc[b]],
                                          gbuf[b], sem_g[b]).wait()

                    def upd(rr, carry2):
                        for c8 in range(d // 16):
                            sl = pl.ds(c8 * 16, 16)
                            gbuf[b][rr, sl] = jnp.maximum(
                                gbuf[b][rr, sl] + hbuf[b][rr, sl], 0.0)
                        return carry2

                    lax.fori_loop(0, C, upd, 0)
                    pltpu.async_copy(gbuf[b], out_hbm.at[pl.ds(base, C)],
                                     sem_o[b])
                    pltpu.make_async_copy(dst_hbm.at[pl.ds(base, C)],
                                          idst[b], sem_d[b]).wait()
                    pltpu.async_copy(gbuf[b], a_sh.at[idst[b]], sem_a[b],
                                     add=True)
                    pltpu.make_async_copy(gbuf[b], out_hbm.at[pl.ds(base, C)],
                                          sem_o[b]).wait()
                    start_in(b, j + K)

            return carry

        lax.fori_loop(0, rounds, rnd, 0)

        for b in range(K):
            jl = (rounds - 1) * K + b

            @pl.when(sid + jl * NS < n_chunks)
            def _():
                pltpu.make_async_copy(gbuf[b], a_sh.at[idst[b]],
                                      sem_a[b]).wait()

        plsc.subcore_barrier()

        def wb(i, carry):
            chunk = sid + i * NS

            @pl.when(chunk < z_chunks)
            def _():
                pltpu.sync_copy(a_sh.at[pl.ds(chunk * ZR, ZR)],
                                a_hbm.at[cid, pl.ds(chunk * ZR, ZR)])

            return carry

        lax.fori_loop(0, z_loop, wb, 0)

    return k(table, src, dst, h, qzero)


def _tc_matmul(x, w, block_rows):
    m, kdim = x.shape
    _, nout = w.shape

    def body(x_ref, w_ref, o_ref):
        o_ref[...] = jnp.dot(x_ref[...], w_ref[...],
                             preferred_element_type=jnp.float32)

    return pl.pallas_call(
        body,
        grid=(m // block_rows,),
        in_specs=[pl.BlockSpec((block_rows, kdim), lambda i: (i, 0)),
                  pl.BlockSpec((kdim, nout), lambda i: (0, 0))],
        out_specs=pl.BlockSpec((block_rows, nout), lambda i: (i, 0)),
        out_shape=jax.ShapeDtypeStruct((m, nout), jnp.float32),
    )(x, w)


def _tc_h0(efeat, w1b, b_init):
    e, de = efeat.shape
    d = w1b.shape[1]
    pk = 128 // de
    ep = e // pk
    wbig = jax.scipy.linalg.block_diag(*([w1b] * pk))
    bbig = jnp.tile(b_init.reshape(-1), pk).reshape(1, pk * d)
    efp = efeat.reshape(ep, pk * de)
    br = 400

    def body(ef_ref, w_ref, b_ref, o_ref):
        o_ref[...] = jnp.dot(ef_ref[...], w_ref[...],
                             preferred_element_type=jnp.float32) + b_ref[...]

    out = pl.pallas_call(
        body,
        grid=(ep // br,),
        in_specs=[pl.BlockSpec((br, pk * de), lambda i: (i, 0)),
                  pl.BlockSpec((pk * de, pk * d), lambda i: (0, 0)),
                  pl.BlockSpec((1, pk * d), lambda i: (0, 0))],
        out_specs=pl.BlockSpec((br, pk * d), lambda i: (i, 0)),
        out_shape=jax.ShapeDtypeStruct((ep, pk * d), jnp.float32),
    )(efp, wbig, bbig)
    return out.reshape(e, d)


def _tc_h(ef, w_upd, b_upd, block_rows):
    e, d = ef.shape
    grid = e // block_rows
    hb = grid // 2

    def body(efa_ref, efb_ref, b_ref, w_ref, o_ref):
        efa = efa_ref[...]
        efb = efb_ref[...]
        o_ref[0, ...] = efa + b_ref[...] - jnp.dot(
            efb, w_ref[...], preferred_element_type=jnp.float32)
        o_ref[1, ...] = efb + b_ref[...] - jnp.dot(
            efa, w_ref[...], preferred_element_type=jnp.float32)

    out = pl.pallas_call(
        body,
        grid=(hb,),
        in_specs=[pl.BlockSpec((block_rows, d), lambda i: (i, 0)),
                  pl.BlockSpec((block_rows, d), lambda i: (i + hb, 0)),
                  pl.BlockSpec((1, d), lambda i: (0, 0)),
                  pl.BlockSpec((d, d), lambda i: (0, 0))],
        out_specs=pl.BlockSpec((2, block_rows, d), lambda i: (0, i, 0)),
        out_shape=jax.ShapeDtypeStruct((2, e // 2, d), jnp.float32),
    )(ef, ef, b_upd, w_upd)
    return out.reshape(e, d)


def _tc_qcomb(a0, a1, w_upd, block_rows):
    n, d = a0.shape

    def body(a0_ref, a1_ref, w_ref, o_ref):
        o_ref[...] = jnp.dot(a0_ref[...] + a1_ref[...], w_ref[...],
                             preferred_element_type=jnp.float32)

    return pl.pallas_call(
        body,
        grid=(n // block_rows,),
        in_specs=[pl.BlockSpec((block_rows, d), lambda i: (i, 0)),
                  pl.BlockSpec((block_rows, d), lambda i: (i, 0)),
                  pl.BlockSpec((d, d), lambda i: (0, 0))],
        out_specs=pl.BlockSpec((block_rows, d), lambda i: (i, 0)),
        out_shape=jax.ShapeDtypeStruct((n, d), jnp.float32),
    )(a0, a1, w_upd)


def _tc_final(nf, m0, m1, wfa, wfb, b_fin, block_rows):
    n, d = nf.shape

    def body(nf_ref, m0_ref, m1_ref, wa_ref, wb_ref, b_ref, o_ref):
        acc = jnp.dot(nf_ref[...], wa_ref[...], preferred_element_type=jnp.float32)
        acc += jnp.dot(m0_ref[...] + m1_ref[...], wb_ref[...],
                       preferred_element_type=jnp.float32)
        o_ref[...] = jnp.maximum(acc + b_ref[...], 0.0)

    return pl.pallas_call(
        body,
        grid=(n // block_rows,),
        in_specs=[pl.BlockSpec((block_rows, d), lambda i: (i, 0)),
                  pl.BlockSpec((block_rows, d), lambda i: (i, 0)),
                  pl.BlockSpec((block_rows, d), lambda i: (i, 0)),
                  pl.BlockSpec((d, d), lambda i: (0, 0)),
                  pl.BlockSpec((d, d), lambda i: (0, 0)),
                  pl.BlockSpec((1, d), lambda i: (0, 0))],
        out_specs=pl.BlockSpec((block_rows, d), lambda i: (i, 0)),
        out_shape=jax.ShapeDtypeStruct((n, d), jnp.float32),
    )(nf, m0, m1, wfa, wfb, b_fin)


STEPS = 4
BLOCK_E = 640
BLOCK_N = 2000


def kernel(node_feature, edge_feature, W_init, b_init, W_upd, b_upd,
           W_fin, b_fin, edge_src, edge_dst):
    n, d = node_feature.shape

    w1a, w1b = W_init[:d], W_init[d:]
    wfa, wfb = W_fin[:d], W_fin[d:]
    b_init2 = b_init.reshape(1, -1)
    b_upd2 = b_upd.reshape(1, -1)
    b_fin2 = b_fin.reshape(1, -1)
    qzero = jnp.zeros((n, d), jnp.float32)

    p = _tc_matmul(node_feature, w1a, BLOCK_N)
    h0 = _tc_h0(edge_feature, w1b, b_init)
    ef, parts = _sc_fused_step(p, edge_src, edge_dst, h0, qzero)

    for _ in range(STEPS):
        q = _tc_qcomb(parts[0], parts[1], W_upd, BLOCK_N)
        hh = _tc_h(ef, W_upd, b_upd2, BLOCK_E)
        ef, parts = _sc_fused_step(q, edge_src, edge_dst, hh, qzero)

    return _tc_final(node_feature, parts[0], parts[1], wfa, wfb, b_fin2, BLOCK_N)


# speedup vs baseline: 1.2737x; 1.2428x over previous
import functools

import jax
import jax.numpy as jnp
from jax import lax
from jax.experimental import pallas as pl
from jax.experimental.pallas import tpu as pltpu
from jax.experimental.pallas import tpu_sc as plsc

NC = 2
NS = 16
C = 64
K = 3
ZR = 200


def _sc_fused_step(table, src, dst, h, qzero):
    e, d = h.shape
    n = table.shape[0]
    eh = e // 2
    n_chunks = eh // C
    nloop = -(-n_chunks // NS)
    rounds = -(-nloop // K)
    z_chunks = n // ZR
    z_loop = -(-z_chunks // NS)

    scratch = ([pltpu.VMEM((C,), jnp.int32) for _ in range(2 * K)]
               + [pltpu.VMEM((C, d), jnp.float32) for _ in range(2 * K)]
               + [pltpu.VMEM_SHARED((n, d), jnp.float32)]
               + [pltpu.SemaphoreType.DMA for _ in range(6 * K)])

    @functools.partial(
        pl.kernel,
        out_type=[jax.ShapeDtypeStruct((e, d), jnp.float32),
                  jax.ShapeDtypeStruct((NC, n, d), jnp.float32)],
        mesh=plsc.VectorSubcoreMesh(core_axis_name="c", subcore_axis_name="s"),
        scratch_types=scratch,
    )
    def k(table_hbm, src_hbm, dst_hbm, h_hbm, qz_hbm, out_hbm, a_hbm, *sc):
        isrc = sc[0:K]
        idst = sc[K:2 * K]
        gbuf = sc[2 * K:3 * K]
        hbuf = sc[3 * K:4 * K]
        a_sh = sc[4 * K]
        sem_s = sc[4 * K + 1:5 * K + 1]
        sem_d = sc[5 * K + 1:6 * K + 1]
        sem_g = sc[6 * K + 1:7 * K + 1]
        sem_h = sc[7 * K + 1:8 * K + 1]
        sem_o = sc[8 * K + 1:9 * K + 1]
        sem_a = sc[9 * K + 1:10 * K + 1]
        cid = lax.axis_index("c")
        sid = lax.axis_index("s")
        ebase = cid * eh

        def zero(i, carry):
            chunk = sid + i * NS

            @pl.when(chunk < z_chunks)
            def _():
                pltpu.sync_copy(qz_hbm.at[pl.ds(chunk * ZR, ZR)],
                                a_sh.at[pl.ds(chunk * ZR, ZR)])

            return carry

        lax.fori_loop(0, z_loop, zero, 0)
        plsc.subcore_barrier()

        def start_in(b, j):
            @pl.when(sid + j * NS < n_chunks)
            def _():
                base = ebase + (sid + j * NS) * C
                pltpu.async_copy(src_hbm.at[pl.ds(base, C)], isrc[b], sem_s[b])
                pltpu.async_copy(dst_hbm.at[pl.ds(base, C)], idst[b], sem_d[b])
                pltpu.async_copy(h_hbm.at[pl.ds(base, C)], hbuf[b], sem_h[b])

        for b in range(K):
            start_in(b, b)

        def rnd(r, carry):
            for b in range(K):
                j = r * K + b

                @pl.when((j >= K) & (sid + (j - K) * NS < n_chunks))
                def _():
                    pltpu.make_async_copy(gbuf[b], a_sh.at[idst[b]],
                                          sem_a[b]).wait()

                @pl.when(sid + j * NS < n_chunks)
                def _():
                    base = ebase + (sid + j * NS) * C
                    pltpu.make_async_copy(src_hbm.at[pl.ds(base, C)],
                                          isrc[b], sem_s[b]).wait()
                    pltpu.async_copy(table_hbm.at[isrc[b]], gbuf[b], sem_g[b])
                    pltpu.make_async_copy(h_hbm.at[pl.ds(base, C)],
                                          hbuf[b], sem_h[b]).wait()
                    pltpu.make_async_copy(table_hbm.at[isrc[b]],
                                          gbuf[b], sem_g[b]).wait()

                    def upd(rr, carry2):
                        for c8 in range(d // 16):
                            sl = pl.ds(c8 * 16, 16)
                            gbuf[b][rr, sl] = jnp.maximum(
                                gbuf[b][rr, sl] + hbuf[b][rr, sl], 0.0)
                        return carry2

                    lax.fori_loop(0, C, upd, 0)
                    pltpu.async_copy(gbuf[b], out_hbm.at[pl.ds(base, C)],
                                     sem_o[b])
                    pltpu.make_async_copy(dst_hbm.at[pl.ds(base, C)],
                                          idst[b], sem_d[b]).wait()
                    pltpu.async_copy(gbuf[b], a_sh.at[idst[b]], sem_a[b],
                                     add=True)
                    pltpu.make_async_copy(gbuf[b], out_hbm.at[pl.ds(base, C)],
                                          sem_o[b]).wait()
                    start_in(b, j + K)

            return carry

        lax.fori_loop(0, rounds, rnd, 0)

        for b in range(K):
            jl = (rounds - 1) * K + b

            @pl.when(sid + jl * NS < n_chunks)
            def _():
                pltpu.make_async_copy(gbuf[b], a_sh.at[idst[b]],
                                      sem_a[b]).wait()

        plsc.subcore_barrier()

        def wb(i, carry):
            chunk = sid + i * NS

            @pl.when(chunk < z_chunks)
            def _():
                pltpu.sync_copy(a_sh.at[pl.ds(chunk * ZR, ZR)],
                                a_hbm.at[cid, pl.ds(chunk * ZR, ZR)])

            return carry

        lax.fori_loop(0, z_loop, wb, 0)

    return k(table, src, dst, h, qzero)


def _tc_matmul(x, w, block_rows):
    m, kdim = x.shape
    _, nout = w.shape

    def body(x_ref, w_ref, o_ref):
        o_ref[...] = jnp.dot(x_ref[...], w_ref[...],
                             preferred_element_type=jnp.float32)

    return pl.pallas_call(
        body,
        grid=(m // block_rows,),
        in_specs=[pl.BlockSpec((block_rows, kdim), lambda i: (i, 0)),
                  pl.BlockSpec((kdim, nout), lambda i: (0, 0))],
        out_specs=pl.BlockSpec((block_rows, nout), lambda i: (i, 0)),
        out_shape=jax.ShapeDtypeStruct((m, nout), jnp.float32),
    )(x, w)


def _tc_h0(efeat, w1b, b_init):
    e, de = efeat.shape
    d = w1b.shape[1]
    pk = 128 // de
    ep = e // pk
    wbig = jax.scipy.linalg.block_diag(*([w1b] * pk))
    bbig = jnp.tile(b_init.reshape(-1), pk).reshape(1, pk * d)
    efp = efeat.reshape(ep, pk * de)
    br = 400

    def body(ef_ref, w_ref, b_ref, o_ref):
        o_ref[...] = jnp.dot(ef_ref[...], w_ref[...],
                             preferred_element_type=jnp.float32) + b_ref[...]

    out = pl.pallas_call(
        body,
        grid=(ep // br,),
        in_specs=[pl.BlockSpec((br, pk * de), lambda i: (i, 0)),
                  pl.BlockSpec((pk * de, pk * d), lambda i: (0, 0)),
                  pl.BlockSpec((1, pk * d), lambda i: (0, 0))],
        out_specs=pl.BlockSpec((br, pk * d), lambda i: (i, 0)),
        out_shape=jax.ShapeDtypeStruct((ep, pk * d), jnp.float32),
    )(efp, wbig, bbig)
    return out.reshape(e, d)


def _tc_h(ef, w_upd, b_upd, block_rows):
    e, d = ef.shape
    grid = e // block_rows
    hb = grid // 2

    def body(efa_ref, efb_ref, b_ref, w_ref, o_ref):
        efa = efa_ref[...]
        efb = efb_ref[...]
        o_ref[0, ...] = efa + b_ref[...] - jnp.dot(
            efb, w_ref[...], preferred_element_type=jnp.float32)
        o_ref[1, ...] = efb + b_ref[...] - jnp.dot(
            efa, w_ref[...], preferred_element_type=jnp.float32)

    out = pl.pallas_call(
        body,
        grid=(hb,),
        in_specs=[pl.BlockSpec((block_rows, d), lambda i: (i, 0)),
                  pl.BlockSpec((block_rows, d), lambda i: (i + hb, 0)),
                  pl.BlockSpec((1, d), lambda i: (0, 0)),
                  pl.BlockSpec((d, d), lambda i: (0, 0))],
        out_specs=pl.BlockSpec((2, block_rows, d), lambda i: (0, i, 0)),
        out_shape=jax.ShapeDtypeStruct((2, e // 2, d), jnp.float32),
    )(ef, ef, b_upd, w_upd)
    return out.reshape(e, d)


def _tc_qcomb(parts, w_upd, block_rows):
    n, d = parts[0].shape
    np_ = len(parts)

    def body(*refs):
        o_ref = refs[-1]
        acc = refs[0][...]
        for r in refs[1:np_]:
            acc = acc + r[...]
        o_ref[...] = jnp.dot(acc, refs[np_][...],
                             preferred_element_type=jnp.float32)

    return pl.pallas_call(
        body,
        grid=(n // block_rows,),
        in_specs=[pl.BlockSpec((block_rows, d), lambda i: (i, 0))
                  for _ in range(np_)]
                 + [pl.BlockSpec((d, d), lambda i: (0, 0))],
        out_specs=pl.BlockSpec((block_rows, d), lambda i: (i, 0)),
        out_shape=jax.ShapeDtypeStruct((n, d), jnp.float32),
    )(*parts, w_upd)


def _tc_final(nf, parts, wfa, wfb, b_fin, block_rows):
    n, d = nf.shape
    np_ = len(parts)

    def body(*refs):
        nf_ref = refs[0]
        o_ref = refs[-1]
        acc = refs[1][...]
        for r in refs[2:1 + np_]:
            acc = acc + r[...]
        out = jnp.dot(nf_ref[...], refs[1 + np_][...],
                      preferred_element_type=jnp.float32)
        out += jnp.dot(acc, refs[2 + np_][...],
                       preferred_element_type=jnp.float32)
        o_ref[...] = jnp.maximum(out + refs[3 + np_][...], 0.0)

    return pl.pallas_call(
        body,
        grid=(n // block_rows,),
        in_specs=[pl.BlockSpec((block_rows, d), lambda i: (i, 0))
                  for _ in range(1 + np_)]
                 + [pl.BlockSpec((d, d), lambda i: (0, 0)),
                    pl.BlockSpec((d, d), lambda i: (0, 0)),
                    pl.BlockSpec((1, d), lambda i: (0, 0))],
        out_specs=pl.BlockSpec((block_rows, d), lambda i: (i, 0)),
        out_shape=jax.ShapeDtypeStruct((n, d), jnp.float32),
    )(nf, *parts, wfa, wfb, b_fin)


STEPS = 4
BLOCK_E = 640
BLOCK_N = 2000


def kernel(node_feature, edge_feature, W_init, b_init, W_upd, b_upd,
           W_fin, b_fin, edge_src, edge_dst):
    n, d = node_feature.shape
    e = edge_src.shape[0]
    e4 = e // 4

    w1a, w1b = W_init[:d], W_init[d:]
    wfa, wfb = W_fin[:d], W_fin[d:]
    b_upd2 = b_upd.reshape(1, -1)
    b_fin2 = b_fin.reshape(1, -1)
    qzero = jnp.zeros((n, d), jnp.float32)

    def regroup(x):
        return (jnp.concatenate([x[0:e4], x[2 * e4:3 * e4]]),
                jnp.concatenate([x[e4:2 * e4], x[3 * e4:4 * e4]]))

    srcg = regroup(edge_src)
    dstg = regroup(edge_dst)
    efg = regroup(edge_feature)

    p = _tc_matmul(node_feature, w1a, BLOCK_N)
    ef = [None, None]
    aa = [None, None]
    for g in range(2):
        h0 = _tc_h0(efg[g], w1b, b_init)
        ef[g], aa[g] = _sc_fused_step(p, srcg[g], dstg[g], h0, qzero)

    for _ in range(STEPS):
        q = _tc_qcomb([aa[0][0], aa[0][1], aa[1][0], aa[1][1]], W_upd, BLOCK_N)
        for g in range(2):
            hh = _tc_h(ef[g], W_upd, b_upd2, BLOCK_E)
            ef[g], aa[g] = _sc_fused_step(q, srcg[g], dstg[g], hh, qzero)

    return _tc_final(node_feature, [aa[0][0], aa[0][1], aa[1][0], aa[1][1]],
                     wfa, wfb, b_fin2, BLOCK_N)


# speedup vs baseline: 1.3229x; 1.0387x over previous
import functools

import jax
import jax.numpy as jnp
from jax import lax
from jax.experimental import pallas as pl
from jax.experimental.pallas import tpu as pltpu
from jax.experimental.pallas import tpu_sc as plsc

NC = 2
NS = 16
C = 80
K = 2
ZR = 200


def _sc_fused_step(table, src, dst, h, qzero):
    e, d = h.shape
    n = table.shape[0]
    eh = e // 2
    n_chunks = eh // C
    nloop = -(-n_chunks // NS)
    rounds = -(-nloop // K)
    z_chunks = n // ZR
    z_loop = -(-z_chunks // NS)

    scratch = ([pltpu.VMEM((C,), jnp.int32) for _ in range(2 * K)]
               + [pltpu.VMEM((C, d), jnp.float32) for _ in range(2 * K)]
               + [pltpu.VMEM_SHARED((n, d), jnp.float32)]
               + [pltpu.SemaphoreType.DMA for _ in range(6 * K)])

    @functools.partial(
        pl.kernel,
        out_type=[jax.ShapeDtypeStruct((e, d), jnp.float32),
                  jax.ShapeDtypeStruct((NC, n, d), jnp.float32)],
        mesh=plsc.VectorSubcoreMesh(core_axis_name="c", subcore_axis_name="s"),
        scratch_types=scratch,
    )
    def k(table_hbm, src_hbm, dst_hbm, h_hbm, qz_hbm, out_hbm, a_hbm, *sc):
        isrc = sc[0:K]
        idst = sc[K:2 * K]
        gbuf = sc[2 * K:3 * K]
        hbuf = sc[3 * K:4 * K]
        a_sh = sc[4 * K]
        sem_s = sc[4 * K + 1:5 * K + 1]
        sem_d = sc[5 * K + 1:6 * K + 1]
        sem_g = sc[6 * K + 1:7 * K + 1]
        sem_h = sc[7 * K + 1:8 * K + 1]
        sem_o = sc[8 * K + 1:9 * K + 1]
        sem_a = sc[9 * K + 1:10 * K + 1]
        cid = lax.axis_index("c")
        sid = lax.axis_index("s")
        ebase = cid * eh

        def zero(i, carry):
            chunk = sid + i * NS

            @pl.when(chunk < z_chunks)
            def _():
                pltpu.sync_copy(qz_hbm.at[pl.ds(chunk * ZR, ZR)],
                                a_sh.at[pl.ds(chunk * ZR, ZR)])

            return carry

        lax.fori_loop(0, z_loop, zero, 0)
        plsc.subcore_barrier()

        def start_in(b, j):
            @pl.when(sid + j * NS < n_chunks)
            def _():
                base = ebase + (sid + j * NS) * C
                pltpu.async_copy(src_hbm.at[pl.ds(base, C)], isrc[b], sem_s[b])
                pltpu.async_copy(dst_hbm.at[pl.ds(base, C)], idst[b], sem_d[b])
                pltpu.async_copy(h_hbm.at[pl.ds(base, C)], hbuf[b], sem_h[b])

        for b in range(K):
            start_in(b, b)

        def rnd(r, carry):
            for b in range(K):
                j = r * K + b

                @pl.when((j >= K) & (sid + (j - K) * NS < n_chunks))
                def _():
                    pltpu.make_async_copy(gbuf[b], a_sh.at[idst[b]],
                                          sem_a[b]).wait()

                @pl.when(sid + j * NS < n_chunks)
                def _():
                    base = ebase + (sid + j * NS) * C
                    pltpu.make_async_copy(src_hbm.at[pl.ds(base, C)],
                                          isrc[b], sem_s[b]).wait()
                    pltpu.async_copy(table_hbm.at[isrc[b]], gbuf[b], sem_g[b])
                    pltpu.make_async_copy(h_hbm.at[pl.ds(base, C)],
                                          hbuf[b], sem_h[b]).wait()
                    pltpu.make_async_copy(table_hbm.at[isrc[b]],
                                          gbuf[b], sem_g[b]).wait()

                    def upd(rr, carry2):
                        for c8 in range(d // 16):
                            sl = pl.ds(c8 * 16, 16)
                            gbuf[b][rr, sl] = jnp.maximum(
                                gbuf[b][rr, sl] + hbuf[b][rr, sl], 0.0)
                        return carry2

                    lax.fori_loop(0, C, upd, 0)
                    pltpu.async_copy(gbuf[b], out_hbm.at[pl.ds(base, C)],
                                     sem_o[b])
                    pltpu.make_async_copy(dst_hbm.at[pl.ds(base, C)],
                                          idst[b], sem_d[b]).wait()
                    pltpu.async_copy(gbuf[b], a_sh.at[idst[b]], sem_a[b],
                                     add=True)
                    pltpu.make_async_copy(gbuf[b], out_hbm.at[pl.ds(base, C)],
                                          sem_o[b]).wait()
                    start_in(b, j + K)

            return carry

        lax.fori_loop(0, rounds, rnd, 0)

        for b in range(K):
            jl = (rounds - 1) * K + b

            @pl.when(sid + jl * NS < n_chunks)
            def _():
                pltpu.make_async_copy(gbuf[b], a_sh.at[idst[b]],
                                      sem_a[b]).wait()

        plsc.subcore_barrier()

        def wb(i, carry):
            chunk = sid + i * NS

            @pl.when(chunk < z_chunks)
            def _():
                pltpu.sync_copy(a_sh.at[pl.ds(chunk * ZR, ZR)],
                                a_hbm.at[cid, pl.ds(chunk * ZR, ZR)])

            return carry

        lax.fori_loop(0, z_loop, wb, 0)

    return k(table, src, dst, h, qzero)


def _tc_matmul(x, w, block_rows):
    m, kdim = x.shape
    _, nout = w.shape

    def body(x_ref, w_ref, o_ref):
        o_ref[...] = jnp.dot(x_ref[...], w_ref[...],
                             preferred_element_type=jnp.float32)

    return pl.pallas_call(
        body,
        grid=(m // block_rows,),
        in_specs=[pl.BlockSpec((block_rows, kdim), lambda i: (i, 0)),
                  pl.BlockSpec((kdim, nout), lambda i: (0, 0))],
        out_specs=pl.BlockSpec((block_rows, nout), lambda i: (i, 0)),
        out_shape=jax.ShapeDtypeStruct((m, nout), jnp.float32),
    )(x, w)


def _tc_h0(efeat, w1b, b_init):
    e, de = efeat.shape
    d = w1b.shape[1]
    pk = 128 // de
    ep = e // pk
    wbig = jax.scipy.linalg.block_diag(*([w1b] * pk))
    bbig = jnp.tile(b_init.reshape(-1), pk).reshape(1, pk * d)
    efp = efeat.reshape(ep, pk * de)
    br = 400

    def body(ef_ref, w_ref, b_ref, o_ref):
        o_ref[...] = jnp.dot(ef_ref[...], w_ref[...],
                             preferred_element_type=jnp.float32) + b_ref[...]

    out = pl.pallas_call(
        body,
        grid=(ep // br,),
        in_specs=[pl.BlockSpec((br, pk * de), lambda i: (i, 0)),
                  pl.BlockSpec((pk * de, pk * d), lambda i: (0, 0)),
                  pl.BlockSpec((1, pk * d), lambda i: (0, 0))],
        out_specs=pl.BlockSpec((br, pk * d), lambda i: (i, 0)),
        out_shape=jax.ShapeDtypeStruct((ep, pk * d), jnp.float32),
    )(efp, wbig, bbig)
    return out.reshape(e, d)


def _tc_h(ef, w_upd, b_upd, block_rows):
    e, d = ef.shape
    grid = e // block_rows
    hb = grid // 2

    def body(efa_ref, efb_ref, b_ref, w_ref, o_ref):
        efa = efa_ref[...]
        efb = efb_ref[...]
        o_ref[0, ...] = efa + b_ref[...] - jnp.dot(
            efb, w_ref[...], preferred_element_type=jnp.float32)
        o_ref[1, ...] = efb + b_ref[...] - jnp.dot(
            efa, w_ref[...], preferred_element_type=jnp.float32)

    out = pl.pallas_call(
        body,
        grid=(hb,),
        in_specs=[pl.BlockSpec((block_rows, d), lambda i: (i, 0)),
                  pl.BlockSpec((block_rows, d), lambda i: (i + hb, 0)),
                  pl.BlockSpec((1, d), lambda i: (0, 0)),
                  pl.BlockSpec((d, d), lambda i: (0, 0))],
        out_specs=pl.BlockSpec((2, block_rows, d), lambda i: (0, i, 0)),
        out_shape=jax.ShapeDtypeStruct((2, e // 2, d), jnp.float32),
    )(ef, ef, b_upd, w_upd)
    return out.reshape(e, d)


def _tc_qcomb(parts, w_upd, block_rows):
    n, d = parts[0].shape
    np_ = len(parts)

    def body(*refs):
        o_ref = refs[-1]
        acc = refs[0][...]
        for r in refs[1:np_]:
            acc = acc + r[...]
        o_ref[...] = jnp.dot(acc, refs[np_][...],
                             preferred_element_type=jnp.float32)

    return pl.pallas_call(
        body,
        grid=(n // block_rows,),
        in_specs=[pl.BlockSpec((block_rows, d), lambda i: (i, 0))
                  for _ in range(np_)]
                 + [pl.BlockSpec((d, d), lambda i: (0, 0))],
        out_specs=pl.BlockSpec((block_rows, d), lambda i: (i, 0)),
        out_shape=jax.ShapeDtypeStruct((n, d), jnp.float32),
    )(*parts, w_upd)


def _tc_final(nf, parts, wfa, wfb, b_fin, block_rows):
    n, d = nf.shape
    np_ = len(parts)

    def body(*refs):
        nf_ref = refs[0]
        o_ref = refs[-1]
        acc = refs[1][...]
        for r in refs[2:1 + np_]:
            acc = acc + r[...]
        out = jnp.dot(nf_ref[...], refs[1 + np_][...],
                      preferred_element_type=jnp.float32)
        out += jnp.dot(acc, refs[2 + np_][...],
                       preferred_element_type=jnp.float32)
        o_ref[...] = jnp.maximum(out + refs[3 + np_][...], 0.0)

    return pl.pallas_call(
        body,
        grid=(n // block_rows,),
        in_specs=[pl.BlockSpec((block_rows, d), lambda i: (i, 0))
                  for _ in range(1 + np_)]
                 + [pl.BlockSpec((d, d), lambda i: (0, 0)),
                    pl.BlockSpec((d, d), lambda i: (0, 0)),
                    pl.BlockSpec((1, d), lambda i: (0, 0))],
        out_specs=pl.BlockSpec((block_rows, d), lambda i: (i, 0)),
        out_shape=jax.ShapeDtypeStruct((n, d), jnp.float32),
    )(nf, *parts, wfa, wfb, b_fin)


STEPS = 4
BLOCK_E = 640
BLOCK_N = 2000


def kernel(node_feature, edge_feature, W_init, b_init, W_upd, b_upd,
           W_fin, b_fin, edge_src, edge_dst):
    n, d = node_feature.shape
    e = edge_src.shape[0]
    e4 = e // 4

    w1a, w1b = W_init[:d], W_init[d:]
    wfa, wfb = W_fin[:d], W_fin[d:]
    b_upd2 = b_upd.reshape(1, -1)
    b_fin2 = b_fin.reshape(1, -1)
    qzero = jnp.zeros((n, d), jnp.float32)

    def regroup(x):
        return (jnp.concatenate([x[0:e4], x[2 * e4:3 * e4]]),
                jnp.concatenate([x[e4:2 * e4], x[3 * e4:4 * e4]]))

    srcg = regroup(edge_src)
    dstg = regroup(edge_dst)
    efg = regroup(edge_feature)

    p = _tc_matmul(node_feature, w1a, BLOCK_N)
    ef = [None, None]
    aa = [None, None]
    for g in range(2):
        h0 = _tc_h0(efg[g], w1b, b_init)
        ef[g], aa[g] = _sc_fused_step(p, srcg[g], dstg[g], h0, qzero)

    for _ in range(STEPS):
        q = _tc_qcomb([aa[0][0], aa[0][1], aa[1][0], aa[1][1]], W_upd, BLOCK_N)
        for g in range(2):
            hh = _tc_h(ef[g], W_upd, b_upd2, BLOCK_E)
            ef[g], aa[g] = _sc_fused_step(q, srcg[g], dstg[g], hh, qzero)

    return _tc_final(node_feature, [aa[0][0], aa[0][1], aa[1][0], aa[1][1]],
                     wfa, wfb, b_fin2, BLOCK_N)


# speedup vs baseline: 1.4058x; 1.0626x over previous
import functools

import jax
import jax.numpy as jnp
from jax import lax
from jax.experimental import pallas as pl
from jax.experimental.pallas import tpu as pltpu
from jax.experimental.pallas import tpu_sc as plsc

NC = 2
NS = 16
C = 80
K = 2
ZR = 200


def _sc_fused_step(table, src, dst, h, qzero):
    e, d = h.shape
    n = table.shape[0]
    eh = e // 2
    n_chunks = eh // C
    nloop = -(-n_chunks // NS)
    rounds = -(-nloop // K)
    z_chunks = n // ZR
    z_loop = -(-z_chunks // NS)

    scratch = ([pltpu.VMEM((C,), jnp.int32) for _ in range(2 * K)]
               + [pltpu.VMEM((C, d), jnp.float32) for _ in range(2 * K)]
               + [pltpu.VMEM_SHARED((n, d), jnp.float32)]
               + [pltpu.SemaphoreType.DMA for _ in range(6 * K)])

    @functools.partial(
        pl.kernel,
        out_type=[jax.ShapeDtypeStruct((e, d), jnp.float32),
                  jax.ShapeDtypeStruct((NC, n, d), jnp.float32)],
        mesh=plsc.VectorSubcoreMesh(core_axis_name="c", subcore_axis_name="s"),
        scratch_types=scratch,
    )
    def k(table_hbm, src_hbm, dst_hbm, h_hbm, qz_hbm, out_hbm, a_hbm, *sc):
        isrc = sc[0:K]
        idst = sc[K:2 * K]
        gbuf = sc[2 * K:3 * K]
        hbuf = sc[3 * K:4 * K]
        a_sh = sc[4 * K]
        sem_s = sc[4 * K + 1:5 * K + 1]
        sem_d = sc[5 * K + 1:6 * K + 1]
        sem_g = sc[6 * K + 1:7 * K + 1]
        sem_h = sc[7 * K + 1:8 * K + 1]
        sem_o = sc[8 * K + 1:9 * K + 1]
        sem_a = sc[9 * K + 1:10 * K + 1]
        cid = lax.axis_index("c")
        sid = lax.axis_index("s")
        ebase = cid * eh

        def zero(i, carry):
            chunk = sid + i * NS

            @pl.when(chunk < z_chunks)
            def _():
                pltpu.sync_copy(qz_hbm.at[pl.ds(chunk * ZR, ZR)],
                                a_sh.at[pl.ds(chunk * ZR, ZR)])

            return carry

        lax.fori_loop(0, z_loop, zero, 0)
        plsc.subcore_barrier()

        def start_in(b, j):
            @pl.when(sid + j * NS < n_chunks)
            def _():
                base = ebase + (sid + j * NS) * C
                pltpu.async_copy(src_hbm.at[pl.ds(base, C)], isrc[b], sem_s[b])
                pltpu.async_copy(dst_hbm.at[pl.ds(base, C)], idst[b], sem_d[b])
                pltpu.async_copy(h_hbm.at[pl.ds(base, C)], hbuf[b], sem_h[b])

        for b in range(K):
            start_in(b, b)

        def rnd(r, carry):
            for b in range(K):
                j = r * K + b

                @pl.when((j >= K) & (sid + (j - K) * NS < n_chunks))
                def _():
                    pltpu.make_async_copy(gbuf[b], a_sh.at[idst[b]],
                                          sem_a[b]).wait()

                @pl.when(sid + j * NS < n_chunks)
                def _():
                    base = ebase + (sid + j * NS) * C
                    pltpu.make_async_copy(src_hbm.at[pl.ds(base, C)],
                                          isrc[b], sem_s[b]).wait()
                    pltpu.async_copy(table_hbm.at[isrc[b]], gbuf[b], sem_g[b])
                    pltpu.make_async_copy(h_hbm.at[pl.ds(base, C)],
                                          hbuf[b], sem_h[b]).wait()
                    pltpu.make_async_copy(table_hbm.at[isrc[b]],
                                          gbuf[b], sem_g[b]).wait()

                    def upd(rr, carry2):
                        for c8 in range(d // 16):
                            sl = pl.ds(c8 * 16, 16)
                            gbuf[b][rr, sl] = jnp.maximum(
                                gbuf[b][rr, sl] + hbuf[b][rr, sl], 0.0)
                        return carry2

                    lax.fori_loop(0, C, upd, 0)
                    pltpu.async_copy(gbuf[b], out_hbm.at[pl.ds(base, C)],
                                     sem_o[b])
                    pltpu.make_async_copy(dst_hbm.at[pl.ds(base, C)],
                                          idst[b], sem_d[b]).wait()
                    pltpu.async_copy(gbuf[b], a_sh.at[idst[b]], sem_a[b],
                                     add=True)
                    pltpu.make_async_copy(gbuf[b], out_hbm.at[pl.ds(base, C)],
                                          sem_o[b]).wait()
                    start_in(b, j + K)

            return carry

        lax.fori_loop(0, rounds, rnd, 0)

        for b in range(K):
            jl = (rounds - 1) * K + b

            @pl.when(sid + jl * NS < n_chunks)
            def _():
                pltpu.make_async_copy(gbuf[b], a_sh.at[idst[b]],
                                      sem_a[b]).wait()

        plsc.subcore_barrier()

        def wb(i, carry):
            chunk = sid + i * NS

            @pl.when(chunk < z_chunks)
            def _():
                pltpu.sync_copy(a_sh.at[pl.ds(chunk * ZR, ZR)],
                                a_hbm.at[cid, pl.ds(chunk * ZR, ZR)])

            return carry

        lax.fori_loop(0, z_loop, wb, 0)

    return k(table, src, dst, h, qzero)


def _tc_matmul(x, w, block_rows):
    m, kdim = x.shape
    _, nout = w.shape

    def body(x_ref, w_ref, o_ref):
        o_ref[...] = jnp.dot(x_ref[...], w_ref[...],
                             preferred_element_type=jnp.float32)

    return pl.pallas_call(
        body,
        grid=(m // block_rows,),
        in_specs=[pl.BlockSpec((block_rows, kdim), lambda i: (i, 0)),
                  pl.BlockSpec((kdim, nout), lambda i: (0, 0))],
        out_specs=pl.BlockSpec((block_rows, nout), lambda i: (i, 0)),
        out_shape=jax.ShapeDtypeStruct((m, nout), jnp.float32),
    )(x, w)


def _tc_h0(efeat, w1b, b_init):
    e, de = efeat.shape
    d = w1b.shape[1]
    pk = 128 // de
    ep = e // pk
    wbig = jax.scipy.linalg.block_diag(*([w1b] * pk))
    bbig = jnp.tile(b_init.reshape(-1), pk).reshape(1, pk * d)
    efp = efeat.reshape(ep, pk * de)
    br = 400

    def body(ef_ref, w_ref, b_ref, o_ref):
        acc = jnp.dot(ef_ref[...], w_ref[...],
                      preferred_element_type=jnp.float32) + b_ref[...]
        o_ref[...] = acc.reshape(br * pk, d)

    return pl.pallas_call(
        body,
        grid=(ep // br,),
        in_specs=[pl.BlockSpec((br, pk * de), lambda i: (i, 0)),
                  pl.BlockSpec((pk * de, pk * d), lambda i: (0, 0)),
                  pl.BlockSpec((1, pk * d), lambda i: (0, 0))],
        out_specs=pl.BlockSpec((br * pk, d), lambda i: (i, 0)),
        out_shape=jax.ShapeDtypeStruct((e, d), jnp.float32),
    )(efp, wbig, bbig)


def _tc_h(ef, w_upd, b_upd, block_rows):
    e, d = ef.shape
    grid = e // block_rows
    hb = grid // 2

    def body(efa_ref, efb_ref, b_ref, w_ref, o_ref):
        efa = efa_ref[...]
        efb = efb_ref[...]
        o_ref[0, ...] = efa + b_ref[...] - jnp.dot(
            efb, w_ref[...], preferred_element_type=jnp.float32)
        o_ref[1, ...] = efb + b_ref[...] - jnp.dot(
            efa, w_ref[...], preferred_element_type=jnp.float32)

    out = pl.pallas_call(
        body,
        grid=(hb,),
        in_specs=[pl.BlockSpec((block_rows, d), lambda i: (i, 0)),
                  pl.BlockSpec((block_rows, d), lambda i: (i + hb, 0)),
                  pl.BlockSpec((1, d), lambda i: (0, 0)),
                  pl.BlockSpec((d, d), lambda i: (0, 0))],
        out_specs=pl.BlockSpec((2, block_rows, d), lambda i: (0, i, 0)),
        out_shape=jax.ShapeDtypeStruct((2, e // 2, d), jnp.float32),
    )(ef, ef, b_upd, w_upd)
    return out.reshape(e, d)


def _tc_qcomb(parts, w_upd, block_rows):
    n, d = parts[0].shape
    np_ = len(parts)

    def body(*refs):
        o_ref = refs[-1]
        acc = refs[0][...]
        for r in refs[1:np_]:
            acc = acc + r[...]
        o_ref[...] = jnp.dot(acc, refs[np_][...],
                             preferred_element_type=jnp.float32)

    return pl.pallas_call(
        body,
        grid=(n // block_rows,),
        in_specs=[pl.BlockSpec((block_rows, d), lambda i: (i, 0))
                  for _ in range(np_)]
                 + [pl.BlockSpec((d, d), lambda i: (0, 0))],
        out_specs=pl.BlockSpec((block_rows, d), lambda i: (i, 0)),
        out_shape=jax.ShapeDtypeStruct((n, d), jnp.float32),
    )(*parts, w_upd)


def _tc_final(nf, parts, wfa, wfb, b_fin, block_rows):
    n, d = nf.shape
    np_ = len(parts)

    def body(*refs):
        nf_ref = refs[0]
        o_ref = refs[-1]
        acc = refs[1][...]
        for r in refs[2:1 + np_]:
            acc = acc + r[...]
        out = jnp.dot(nf_ref[...], refs[1 + np_][...],
                      preferred_element_type=jnp.float32)
        out += jnp.dot(acc, refs[2 + np_][...],
                       preferred_element_type=jnp.float32)
        o_ref[...] = jnp.maximum(out + refs[3 + np_][...], 0.0)

    return pl.pallas_call(
        body,
        grid=(n // block_rows,),
        in_specs=[pl.BlockSpec((block_rows, d), lambda i: (i, 0))
                  for _ in range(1 + np_)]
                 + [pl.BlockSpec((d, d), lambda i: (0, 0)),
                    pl.BlockSpec((d, d), lambda i: (0, 0)),
                    pl.BlockSpec((1, d), lambda i: (0, 0))],
        out_specs=pl.BlockSpec((block_rows, d), lambda i: (i, 0)),
        out_shape=jax.ShapeDtypeStruct((n, d), jnp.float32),
    )(nf, *parts, wfa, wfb, b_fin)


STEPS = 4
BLOCK_E = 640
BLOCK_N = 2000


def kernel(node_feature, edge_feature, W_init, b_init, W_upd, b_upd,
           W_fin, b_fin, edge_src, edge_dst):
    n, d = node_feature.shape
    e = edge_src.shape[0]
    e4 = e // 4

    w1a, w1b = W_init[:d], W_init[d:]
    wfa, wfb = W_fin[:d], W_fin[d:]
    b_upd2 = b_upd.reshape(1, -1)
    b_fin2 = b_fin.reshape(1, -1)
    qzero = jnp.zeros((n, d), jnp.float32)

    def regroup(x):
        return (jnp.concatenate([x[0:e4], x[2 * e4:3 * e4]]),
                jnp.concatenate([x[e4:2 * e4], x[3 * e4:4 * e4]]))

    srcg = regroup(edge_src)
    dstg = regroup(edge_dst)
    efg = regroup(edge_feature)

    p = _tc_matmul(node_feature, w1a, BLOCK_N)
    ef = [None, None]
    aa = [None, None]
    for g in range(2):
        h0 = _tc_h0(efg[g], w1b, b_init)
        ef[g], aa[g] = _sc_fused_step(p, srcg[g], dstg[g], h0, qzero)

    for _ in range(STEPS):
        q = _tc_qcomb([aa[0][0], aa[0][1], aa[1][0], aa[1][1]], W_upd, BLOCK_N)
        for g in range(2):
            hh = _tc_h(ef[g], W_upd, b_upd2, BLOCK_E)
            ef[g], aa[g] = _sc_fused_step(q, srcg[g], dstg[g], hh, qzero)

    return _tc_final(node_feature, [aa[0][0], aa[0][1], aa[1][0], aa[1][1]],
                     wfa, wfb, b_fin2, BLOCK_N)


# speedup vs baseline: 1.4115x; 1.0041x over previous
import functools

import jax
import jax.numpy as jnp
from jax import lax
from jax.experimental import pallas as pl
from jax.experimental.pallas import tpu as pltpu
from jax.experimental.pallas import tpu_sc as plsc

NC = 2
NS = 16
C = 80
K = 2
ZR = 200


def _sc_fused_step(table, src, dst, h, qzero):
    e, d = h.shape
    n = table.shape[0]
    eh = e // 2
    n_chunks = eh // C
    nloop = -(-n_chunks // NS)
    rounds = -(-nloop // K)
    z_chunks = n // ZR
    z_loop = -(-z_chunks // NS)

    scratch = ([pltpu.VMEM((C,), jnp.int32) for _ in range(2 * K)]
               + [pltpu.VMEM((C, d), jnp.float32) for _ in range(2 * K)]
               + [pltpu.VMEM_SHARED((n, d), jnp.float32)]
               + [pltpu.SemaphoreType.DMA for _ in range(6 * K)])

    @functools.partial(
        pl.kernel,
        out_type=[jax.ShapeDtypeStruct((e, d), jnp.float32),
                  jax.ShapeDtypeStruct((NC, n, d), jnp.float32)],
        mesh=plsc.VectorSubcoreMesh(core_axis_name="c", subcore_axis_name="s"),
        scratch_types=scratch,
    )
    def k(table_hbm, src_hbm, dst_hbm, h_hbm, qz_hbm, out_hbm, a_hbm, *sc):
        isrc = sc[0:K]
        idst = sc[K:2 * K]
        gbuf = sc[2 * K:3 * K]
        hbuf = sc[3 * K:4 * K]
        a_sh = sc[4 * K]
        sem_s = sc[4 * K + 1:5 * K + 1]
        sem_d = sc[5 * K + 1:6 * K + 1]
        sem_g = sc[6 * K + 1:7 * K + 1]
        sem_h = sc[7 * K + 1:8 * K + 1]
        sem_o = sc[8 * K + 1:9 * K + 1]
        sem_a = sc[9 * K + 1:10 * K + 1]
        cid = lax.axis_index("c")
        sid = lax.axis_index("s")
        ebase = cid * eh

        def zero(i, carry):
            chunk = sid + i * NS

            @pl.when(chunk < z_chunks)
            def _():
                pltpu.sync_copy(qz_hbm.at[pl.ds(chunk * ZR, ZR)],
                                a_sh.at[pl.ds(chunk * ZR, ZR)])

            return carry

        lax.fori_loop(0, z_loop, zero, 0)
        plsc.subcore_barrier()

        def start_in(b, j):
            @pl.when(sid + j * NS < n_chunks)
            def _():
                base = ebase + (sid + j * NS) * C
                pltpu.async_copy(src_hbm.at[pl.ds(base, C)], isrc[b], sem_s[b])
                pltpu.async_copy(dst_hbm.at[pl.ds(base, C)], idst[b], sem_d[b])
                pltpu.async_copy(h_hbm.at[pl.ds(base, C)], hbuf[b], sem_h[b])

        for b in range(K):
            start_in(b, b)

        def rnd(r, carry):
            for b in range(K):
                j = r * K + b

                @pl.when((j >= K) & (sid + (j - K) * NS < n_chunks))
                def _():
                    pltpu.make_async_copy(gbuf[b], a_sh.at[idst[b]],
                                          sem_a[b]).wait()

                @pl.when(sid + j * NS < n_chunks)
                def _():
                    base = ebase + (sid + j * NS) * C
                    pltpu.make_async_copy(src_hbm.at[pl.ds(base, C)],
                                          isrc[b], sem_s[b]).wait()
                    pltpu.async_copy(table_hbm.at[isrc[b]], gbuf[b], sem_g[b])
                    pltpu.make_async_copy(h_hbm.at[pl.ds(base, C)],
                                          hbuf[b], sem_h[b]).wait()
                    pltpu.make_async_copy(table_hbm.at[isrc[b]],
                                          gbuf[b], sem_g[b]).wait()

                    def upd(rr, carry2):
                        for c8 in range(d // 16):
                            sl = pl.ds(c8 * 16, 16)
                            gbuf[b][rr, sl] = jnp.maximum(
                                gbuf[b][rr, sl] + hbuf[b][rr, sl], 0.0)
                        return carry2

                    lax.fori_loop(0, C, upd, 0)
                    pltpu.async_copy(gbuf[b], out_hbm.at[pl.ds(base, C)],
                                     sem_o[b])
                    pltpu.make_async_copy(dst_hbm.at[pl.ds(base, C)],
                                          idst[b], sem_d[b]).wait()
                    pltpu.async_copy(gbuf[b], a_sh.at[idst[b]], sem_a[b],
                                     add=True)
                    pltpu.make_async_copy(gbuf[b], out_hbm.at[pl.ds(base, C)],
                                          sem_o[b]).wait()
                    start_in(b, j + K)

            return carry

        lax.fori_loop(0, rounds, rnd, 0)

        for b in range(K):
            jl = (rounds - 1) * K + b

            @pl.when(sid + jl * NS < n_chunks)
            def _():
                pltpu.make_async_copy(gbuf[b], a_sh.at[idst[b]],
                                      sem_a[b]).wait()

        plsc.subcore_barrier()

        def wb(i, carry):
            chunk = sid + i * NS

            @pl.when(chunk < z_chunks)
            def _():
                pltpu.sync_copy(a_sh.at[pl.ds(chunk * ZR, ZR)],
                                a_hbm.at[cid, pl.ds(chunk * ZR, ZR)])

            return carry

        lax.fori_loop(0, z_loop, wb, 0)

    return k(table, src, dst, h, qzero)


def _tc_matmul(x, w, block_rows):
    m, kdim = x.shape
    _, nout = w.shape

    def body(x_ref, w_ref, o_ref):
        o_ref[...] = jnp.dot(x_ref[...], w_ref[...],
                             preferred_element_type=jnp.float32)

    return pl.pallas_call(
        body,
        grid=(m // block_rows,),
        in_specs=[pl.BlockSpec((block_rows, kdim), lambda i: (i, 0)),
                  pl.BlockSpec((kdim, nout), lambda i: (0, 0))],
        out_specs=pl.BlockSpec((block_rows, nout), lambda i: (i, 0)),
        out_shape=jax.ShapeDtypeStruct((m, nout), jnp.float32),
    )(x, w)


def _tc_h0(efeat, group, w1b, b_init):
    e, de = efeat.shape
    d = w1b.shape[1]
    pk = 128 // de
    br = 400
    rows = br * pk
    qb = (e // 4) // rows
    eg = e // 2
    wbig = jax.scipy.linalg.block_diag(*([w1b] * pk))
    bbig = jnp.tile(b_init.reshape(-1), pk).reshape(1, pk * d)

    efp = efeat.reshape(e // pk, pk * de)

    def imap(i):
        return (jnp.where(i < qb, group * qb + i, (group + 2) * qb + i - qb), 0)

    def body(ef_ref, w_ref, b_ref, o_ref):
        acc = jnp.dot(ef_ref[...], w_ref[...],
                      preferred_element_type=jnp.float32) + b_ref[...]
        o_ref[...] = acc.reshape(rows, d)

    return pl.pallas_call(
        body,
        grid=(eg // rows,),
        in_specs=[pl.BlockSpec((br, pk * de), imap),
                  pl.BlockSpec((pk * de, pk * d), lambda i: (0, 0)),
                  pl.BlockSpec((1, pk * d), lambda i: (0, 0))],
        out_specs=pl.BlockSpec((rows, d), lambda i: (i, 0)),
        out_shape=jax.ShapeDtypeStruct((eg, d), jnp.float32),
    )(efp, wbig, bbig)


def _tc_h(ef, w_upd, b_upd, block_rows):
    e, d = ef.shape
    grid = e // block_rows
    hb = grid // 2

    def body(efa_ref, efb_ref, b_ref, w_ref, o_ref):
        efa = efa_ref[...]
        efb = efb_ref[...]
        o_ref[0, ...] = efa + b_ref[...] - jnp.dot(
            efb, w_ref[...], preferred_element_type=jnp.float32)
        o_ref[1, ...] = efb + b_ref[...] - jnp.dot(
            efa, w_ref[...], preferred_element_type=jnp.float32)

    out = pl.pallas_call(
        body,
        grid=(hb,),
        in_specs=[pl.BlockSpec((block_rows, d), lambda i: (i, 0)),
                  pl.BlockSpec((block_rows, d), lambda i: (i + hb, 0)),
                  pl.BlockSpec((1, d), lambda i: (0, 0)),
                  pl.BlockSpec((d, d), lambda i: (0, 0))],
        out_specs=pl.BlockSpec((2, block_rows, d), lambda i: (0, i, 0)),
        out_shape=jax.ShapeDtypeStruct((2, e // 2, d), jnp.float32),
    )(ef, ef, b_upd, w_upd)
    return out.reshape(e, d)


def _tc_qcomb(parts, w_upd, block_rows):
    n, d = parts[0].shape
    np_ = len(parts)

    def body(*refs):
        o_ref = refs[-1]
        acc = refs[0][...]
        for r in refs[1:np_]:
            acc = acc + r[...]
        o_ref[...] = jnp.dot(acc, refs[np_][...],
                             preferred_element_type=jnp.float32)

    return pl.pallas_call(
        body,
        grid=(n // block_rows,),
        in_specs=[pl.BlockSpec((block_rows, d), lambda i: (i, 0))
                  for _ in range(np_)]
                 + [pl.BlockSpec((d, d), lambda i: (0, 0))],
        out_specs=pl.BlockSpec((block_rows, d), lambda i: (i, 0)),
        out_shape=jax.ShapeDtypeStruct((n, d), jnp.float32),
    )(*parts, w_upd)


def _tc_final(nf, parts, wfa, wfb, b_fin, block_rows):
    n, d = nf.shape
    np_ = len(parts)

    def body(*refs):
        nf_ref = refs[0]
        o_ref = refs[-1]
        acc = refs[1][...]
        for r in refs[2:1 + np_]:
            acc = acc + r[...]
        out = jnp.dot(nf_ref[...], refs[1 + np_][...],
                      preferred_element_type=jnp.float32)
        out += jnp.dot(acc, refs[2 + np_][...],
                       preferred_element_type=jnp.float32)
        o_ref[...] = jnp.maximum(out + refs[3 + np_][...], 0.0)

    return pl.pallas_call(
        body,
        grid=(n // block_rows,),
        in_specs=[pl.BlockSpec((block_rows, d), lambda i: (i, 0))
                  for _ in range(1 + np_)]
                 + [pl.BlockSpec((d, d), lambda i: (0, 0)),
                    pl.BlockSpec((d, d), lambda i: (0, 0)),
                    pl.BlockSpec((1, d), lambda i: (0, 0))],
        out_specs=pl.BlockSpec((block_rows, d), lambda i: (i, 0)),
        out_shape=jax.ShapeDtypeStruct((n, d), jnp.float32),
    )(nf, *parts, wfa, wfb, b_fin)


STEPS = 4
BLOCK_E = 640
BLOCK_N = 2000


def kernel(node_feature, edge_feature, W_init, b_init, W_upd, b_upd,
           W_fin, b_fin, edge_src, edge_dst):
    n, d = node_feature.shape
    e = edge_src.shape[0]
    e4 = e // 4

    w1a, w1b = W_init[:d], W_init[d:]
    wfa, wfb = W_fin[:d], W_fin[d:]
    b_upd2 = b_upd.reshape(1, -1)
    b_fin2 = b_fin.reshape(1, -1)
    qzero = jnp.zeros((n, d), jnp.float32)

    def regroup(x):
        return (jnp.concatenate([x[0:e4], x[2 * e4:3 * e4]]),
                jnp.concatenate([x[e4:2 * e4], x[3 * e4:4 * e4]]))

    srcg = regroup(edge_src)
    dstg = regroup(edge_dst)

    p = _tc_matmul(node_feature, w1a, BLOCK_N)
    ef = [None, None]
    aa = [None, None]
    for g in range(2):
        h0 = _tc_h0(edge_feature, g, w1b, b_init)
        ef[g], aa[g] = _sc_fused_step(p, srcg[g], dstg[g], h0, qzero)

    for _ in range(STEPS):
        q = _tc_qcomb([aa[0][0], aa[0][1], aa[1][0], aa[1][1]], W_upd, BLOCK_N)
        for g in range(2):
            hh = _tc_h(ef[g], W_upd, b_upd2, BLOCK_E)
            ef[g], aa[g] = _sc_fused_step(q, srcg[g], dstg[g], hh, qzero)

    return _tc_final(node_feature, [aa[0][0], aa[0][1], aa[1][0], aa[1][1]],
                     wfa, wfb, b_fin2, BLOCK_N)
